# baseline (speedup 1.0000x reference)
import functools

import jax
import jax.numpy as jnp
from jax import lax
from jax.experimental import pallas as pl
from jax.experimental.pallas import tpu as pltpu
from jax.experimental.pallas import tpu_sc as plsc

R = 128
N = 8192
K = 1024
L = 16
NC = 2
NS = 16
W = NC * NS
RPW = R // W
NV = N // L
BLK = N // L
UNROLL = 4

I32MAX = 0x7FFFFFFF


def _srl(x, n):
    return lax.shift_right_logical(x, n)


def _body(attn_hbm, inp_hbm, oattn_hbm, oinp_hbm,
          attn_v, inp_v, skey_v, ckey_a, cidx_a, ckey_b, cidx_b,
          hist256, hist32, binsum, stag_a, stag_i):
    cid = lax.axis_index("c")
    sid = lax.axis_index("s")
    wid = sid * NC + cid
    lanes = lax.broadcasted_iota(jnp.int32, (L,), 0)
    zeros = jnp.zeros((L,), jnp.int32)
    ones = jnp.ones((L,), jnp.int32)
    lanebase = lanes * 256

    def z0(j, _):
        hist256[pl.ds(j * L, L)] = zeros
        return 0
    lax.fori_loop(0, 256, z0, 0)

    def byte_scan(threshold, save_binsum):
        def grp(j, carry):
            cum_carry, cntv = carry
            acc = zeros
            for l in range(L):
                sl = l * 256 + j * L
                acc = acc + hist256[pl.ds(sl, L)]
                hist256[pl.ds(sl, L)] = zeros
            inc = plsc.cumsum(acc) + cum_carry
            if save_binsum:
                binsum[pl.ds(j * L, L)] = inc
            cntv = cntv + jnp.where(inc < threshold, 1, 0)
            return (jnp.max(inc), cntv)
        _, cntv = lax.fori_loop(0, 16, grp, (jnp.int32(0), zeros))
        return jnp.sum(cntv)

    def do_row(i, _carry):
        r = wid * RPW + i
        pltpu.sync_copy(attn_hbm.at[r], attn_v)
        pltpu.sync_copy(inp_hbm.at[r], inp_v)

        def key_hist(t4, _):
            for u in range(UNROLL):
                t = t4 * UNROLL + u
                x = attn_v[pl.ds(t * L, L)]
                uu = lax.bitcast_convert_type(x, jnp.int32)
                sk = jnp.where(uu < 0, uu & I32MAX, ~uu)
                skey_v[pl.ds(t * L, L)] = sk
                d = (_srl(sk, 24) & 0xFF) ^ 0x80
                slot = lanebase + d
                plsc.addupdate_scatter(hist256, [slot], ones)
            return 0
        lax.fori_loop(0, NV // UNROLL, key_hist, 0)

        p1 = byte_scan(jnp.int32(K), True)
        bm1 = plsc.load_gather(binsum, [zeros + jnp.maximum(p1 - 1, 0)])
        below1 = jnp.where(p1 == 0, jnp.int32(0), jnp.max(bm1))

        def hist2(t4, _):
            for u in range(UNROLL):
                t = t4 * UNROLL + u
                sk = skey_v[pl.ds(t * L, L)]
                d1 = (_srl(sk, 24) & 0xFF) ^ 0x80
                m = d1 == p1
                slot = lanebase + (_srl(sk, 16) & 0xFF)
                plsc.addupdate_scatter(hist256, [slot], ones, mask=m)
            return 0
        lax.fori_loop(0, NV // UNROLL, hist2, 0)

        p2 = byte_scan(K - below1, False)

        skey_ub = (
            lax.shift_left((p1 ^ 0x80), 24)
            | lax.shift_left(p2, 16)
            | jnp.int32(0xFFFF))

        def compact(t4, off):
            for u in range(UNROLL):
                t = t4 * UNROLL + u
                pos = lanes * BLK + t
                s = plsc.load_gather(skey_v, [pos])
                m = s <= skey_ub
                plsc.store_scatter(ckey_a, [off], s, mask=m)
                plsc.store_scatter(cidx_a, [off], pos, mask=m)
                off = off + jnp.where(m, 1, 0)
            return off
        off_fin = lax.fori_loop(0, BLK // UNROLL, compact, lanes * BLK)
        cnt = off_fin - lanes * BLK
        ncand = jnp.sum(cnt)
        cmax = jnp.max(cnt)
        c1 = (ncand + (L - 1)) // L

        def zero32(j, _):
            hist32[pl.ds(j * L, L)] = zeros
            return 0

        def sort_pass(p, src_k, src_i, dst_k, dst_i, span, stride, msk_cnt):
            sh = 5 * p
            flip = 2 if p == 6 else 0
            lax.fori_loop(0, 32, zero32, 0)

            def hist_step(t, _):
                pos = lanes * stride + t
                k = plsc.load_gather(src_k, [pos])
                d = (_srl(k, sh) & 0x1F) ^ flip
                slot = d * L + lanes
                m = None if msk_cnt is None else (t < msk_cnt)
                plsc.addupdate_scatter(hist32, [slot], ones, mask=m)
                return 0
            lax.fori_loop(0, span, hist_step, 0)

            def scan_step(j, carry):
                v = hist32[pl.ds(j * L, L)]
                inc = plsc.cumsum(v)
                hist32[pl.ds(j * L, L)] = inc - v + carry
                return carry + jnp.max(inc)
            lax.fori_loop(0, 32, scan_step, jnp.int32(0))

            def perm_step(t, _):
                pos = lanes * stride + t
                k = plsc.load_gather(src_k, [pos])
                v = plsc.load_gather(src_i, [pos])
                d = (_srl(k, sh) & 0x1F) ^ flip
                slot = d * L + lanes
                o = plsc.load_gather(hist32, [slot])
                m = None if msk_cnt is None else (t < msk_cnt)
                plsc.store_scatter(hist32, [slot], o + 1, mask=m)
                if p == 6:
                    mo = o < K if m is None else (m & (o < K))
                    va = plsc.load_gather(attn_v, [v])
                    vi = plsc.load_gather(inp_v, [v])
                    plsc.store_scatter(stag_a, [o], va, mask=mo)
                    plsc.store_scatter(stag_i, [o], vi, mask=mo)
                else:
                    plsc.store_scatter(dst_k, [o], k, mask=m)
                    plsc.store_scatter(dst_i, [o], v, mask=m)
                return 0
            lax.fori_loop(0, span, perm_step, 0)

        sort_pass(0, ckey_a, cidx_a, ckey_b, cidx_b, cmax, BLK, cnt)

        padpos = ncand + lanes
        padm = padpos < c1 * L
        plsc.store_scatter(
            ckey_b, [padpos], jnp.full((L,), I32MAX, jnp.int32), mask=padm)
        plsc.store_scatter(cidx_b, [padpos], zeros, mask=padm)

        bufs = ((ckey_b, cidx_b), (ckey_a, cidx_a))
        for p in range(1, 7):
            src_k, src_i = bufs[(p - 1) % 2]
            dst_k, dst_i = bufs[p % 2]
            sort_pass(p, src_k, src_i, dst_k, dst_i, c1, c1, None)

        pltpu.sync_copy(stag_a, oattn_hbm.at[r])
        pltpu.sync_copy(stag_i, oinp_hbm.at[r])
        return 0

    lax.fori_loop(0, RPW, do_row, 0)


@functools.partial(jax.jit, static_argnames=("interpret",))
def _run(attn, inputs, interpret=False):
    mesh = plsc.VectorSubcoreMesh(
        core_axis_name="c", subcore_axis_name="s",
        num_cores=NC, num_subcores=NS)
    f = pl.kernel(
        _body,
        out_type=(
            jax.ShapeDtypeStruct((R, K), jnp.float32),
            jax.ShapeDtypeStruct((R, K), jnp.float32),
        ),
        mesh=mesh,
        scratch_types=[
            pltpu.VMEM((N,), jnp.float32),
            pltpu.VMEM((N,), jnp.float32),
            pltpu.VMEM((N,), jnp.int32),
            pltpu.VMEM((N,), jnp.int32),
            pltpu.VMEM((N,), jnp.int32),
            pltpu.VMEM((N,), jnp.int32),
            pltpu.VMEM((N,), jnp.int32),
            pltpu.VMEM((256 * L,), jnp.int32),
            pltpu.VMEM((32 * L,), jnp.int32),
            pltpu.VMEM((256,), jnp.int32),
            pltpu.VMEM((K,), jnp.float32),
            pltpu.VMEM((K,), jnp.float32),
        ],
        compiler_params=pltpu.CompilerParams(needs_layout_passes=False),
        interpret=interpret,
    )
    return f(attn, inputs)


def kernel(attn, inputs):
    return _run(attn, inputs)


# speedup vs baseline: 1.0231x; 1.0231x over previous
import functools

import jax
import jax.numpy as jnp
from jax import lax
from jax.experimental import pallas as pl
from jax.experimental.pallas import tpu as pltpu
from jax.experimental.pallas import tpu_sc as plsc

R = 128
N = 8192
K = 1024
L = 16
NC = 2
NS = 16
W = NC * NS
RPW = R // W
NV = N // L
BLK = N // L
UNROLL = 4

I32MAX = 0x7FFFFFFF


def _srl(x, n):
    return lax.shift_right_logical(x, n)


def _body(attn_hbm, inp_hbm, oattn_hbm, oinp_hbm,
          attn_v, inp_v, skey_v, ckey_a, cidx_a, ckey_b, cidx_b,
          hist256, hist32, hist32b, binsum, stag_a, stag_i):
    cid = lax.axis_index("c")
    sid = lax.axis_index("s")
    wid = sid * NC + cid
    lanes = lax.broadcasted_iota(jnp.int32, (L,), 0)
    zeros = jnp.zeros((L,), jnp.int32)
    ones = jnp.ones((L,), jnp.int32)
    lanebase = lanes * 256

    def z0(j, _):
        hist256[pl.ds(j * L, L)] = zeros
        return 0
    lax.fori_loop(0, 256, z0, 0)

    def byte_scan(threshold, save_binsum):
        def grp(j, carry):
            cum_carry, cntv = carry
            acc = zeros
            for l in range(L):
                sl = l * 256 + j * L
                acc = acc + hist256[pl.ds(sl, L)]
                hist256[pl.ds(sl, L)] = zeros
            inc = plsc.cumsum(acc) + cum_carry
            if save_binsum:
                binsum[pl.ds(j * L, L)] = inc
            cntv = cntv + jnp.where(inc < threshold, 1, 0)
            return (jnp.max(inc), cntv)
        _, cntv = lax.fori_loop(0, 16, grp, (jnp.int32(0), zeros))
        return jnp.sum(cntv)

    def do_row(i, _carry):
        r = wid * RPW + i
        pltpu.sync_copy(attn_hbm.at[r], attn_v)
        pltpu.sync_copy(inp_hbm.at[r], inp_v)

        def key_hist(t4, _):
            for u in range(UNROLL):
                t = t4 * UNROLL + u
                x = attn_v[pl.ds(t * L, L)]
                uu = lax.bitcast_convert_type(x, jnp.int32)
                sk = jnp.where(uu < 0, uu & I32MAX, ~uu)
                skey_v[pl.ds(t * L, L)] = sk
                d = (_srl(sk, 24) & 0xFF) ^ 0x80
                slot = lanebase + d
                plsc.addupdate_scatter(hist256, [slot], ones)
            return 0
        lax.fori_loop(0, NV // UNROLL, key_hist, 0)

        p1 = byte_scan(jnp.int32(K), True)
        bm1 = plsc.load_gather(binsum, [zeros + jnp.maximum(p1 - 1, 0)])
        below1 = jnp.where(p1 == 0, jnp.int32(0), jnp.max(bm1))

        def hist2(t4, _):
            for u in range(UNROLL):
                t = t4 * UNROLL + u
                sk = skey_v[pl.ds(t * L, L)]
                d1 = (_srl(sk, 24) & 0xFF) ^ 0x80
                m = d1 == p1
                slot = lanebase + (_srl(sk, 16) & 0xFF)
                plsc.addupdate_scatter(hist256, [slot], ones, mask=m)
            return 0
        lax.fori_loop(0, NV // UNROLL, hist2, 0)

        p2 = byte_scan(K - below1, False)

        skey_ub = (
            lax.shift_left((p1 ^ 0x80), 24)
            | lax.shift_left(p2, 16)
            | jnp.int32(0xFFFF))

        def zero32(tbl):
            def z(j, _):
                tbl[pl.ds(j * L, L)] = zeros
                return 0
            lax.fori_loop(0, 32, z, 0)

        zero32(hist32)

        def compact(t4, off):
            for u in range(UNROLL):
                t = t4 * UNROLL + u
                pos = lanes * BLK + t
                s = plsc.load_gather(skey_v, [pos])
                m = s <= skey_ub
                plsc.store_scatter(ckey_a, [off], s, mask=m)
                plsc.store_scatter(cidx_a, [off], pos, mask=m)
                plsc.addupdate_scatter(
                    hist32, [(s & 0x1F) * L + lanes], ones, mask=m)
                off = off + jnp.where(m, 1, 0)
            return off
        off_fin = lax.fori_loop(0, BLK // UNROLL, compact, lanes * BLK)
        cnt = off_fin - lanes * BLK
        ncand = jnp.sum(cnt)
        cmax = jnp.max(cnt)
        c1 = (ncand + (L - 1)) // L

        hists = (hist32, hist32b)

        def scan32(tbl):
            def s(j, carry):
                v = tbl[pl.ds(j * L, L)]
                inc = plsc.cumsum(v)
                tbl[pl.ds(j * L, L)] = inc - v + carry
                return carry + jnp.max(inc)
            lax.fori_loop(0, 32, s, jnp.int32(0))

        def sort_pass(p, src_k, src_i, dst_k, dst_i, span, stride, msk_cnt):
            tbl = hists[0]
            sh = 5 * p
            flip = 2 if p == 6 else 0
            if p > 0:
                zero32(tbl)

                def hist_step(t, _):
                    pos = lanes * stride + t
                    k = plsc.load_gather(src_k, [pos])
                    d = (_srl(k, sh) & 0x1F) ^ flip
                    plsc.addupdate_scatter(tbl, [d * L + lanes], ones)
                    return 0
                lax.fori_loop(0, span, hist_step, 0)
            scan32(tbl)

            def perm_step(t, _):
                pos = lanes * stride + t
                k = plsc.load_gather(src_k, [pos])
                v = plsc.load_gather(src_i, [pos])
                d = (_srl(k, sh) & 0x1F) ^ flip
                slot = d * L + lanes
                o = plsc.load_gather(tbl, [slot])
                m = None if msk_cnt is None else (t < msk_cnt)
                plsc.store_scatter(tbl, [slot], o + 1, mask=m)
                if p == 6:
                    mo = o < K if m is None else (m & (o < K))
                    va = plsc.load_gather(attn_v, [v])
                    vi = plsc.load_gather(inp_v, [v])
                    plsc.store_scatter(stag_a, [o], va, mask=mo)
                    plsc.store_scatter(stag_i, [o], vi, mask=mo)
                else:
                    plsc.store_scatter(dst_k, [o], k, mask=m)
                    plsc.store_scatter(dst_i, [o], v, mask=m)
                return 0
            lax.fori_loop(0, span, perm_step, 0)

        sort_pass(0, ckey_a, cidx_a, ckey_b, cidx_b, cmax, BLK, cnt)

        padpos = ncand + lanes
        padm = padpos < c1 * L
        plsc.store_scatter(
            ckey_b, [padpos], jnp.full((L,), I32MAX, jnp.int32), mask=padm)
        plsc.store_scatter(cidx_b, [padpos], zeros, mask=padm)

        bufs = ((ckey_b, cidx_b), (ckey_a, cidx_a))
        for p in range(1, 7):
            src_k, src_i = bufs[(p - 1) % 2]
            dst_k, dst_i = bufs[p % 2]
            sort_pass(p, src_k, src_i, dst_k, dst_i, c1, c1, None)

        pltpu.sync_copy(stag_a, oattn_hbm.at[r])
        pltpu.sync_copy(stag_i, oinp_hbm.at[r])
        return 0

    lax.fori_loop(0, RPW, do_row, 0)


@functools.partial(jax.jit, static_argnames=("interpret",))
def _run(attn, inputs, interpret=False):
    mesh = plsc.VectorSubcoreMesh(
        core_axis_name="c", subcore_axis_name="s",
        num_cores=NC, num_subcores=NS)
    f = pl.kernel(
        _body,
        out_type=(
            jax.ShapeDtypeStruct((R, K), jnp.float32),
            jax.ShapeDtypeStruct((R, K), jnp.float32),
        ),
        mesh=mesh,
        scratch_types=[
            pltpu.VMEM((N,), jnp.float32),
            pltpu.VMEM((N,), jnp.float32),
            pltpu.VMEM((N,), jnp.int32),
            pltpu.VMEM((N,), jnp.int32),
            pltpu.VMEM((N,), jnp.int32),
            pltpu.VMEM((N,), jnp.int32),
            pltpu.VMEM((N,), jnp.int32),
            pltpu.VMEM((256 * L,), jnp.int32),
            pltpu.VMEM((32 * L,), jnp.int32),
            pltpu.VMEM((32 * L,), jnp.int32),
            pltpu.VMEM((256,), jnp.int32),
            pltpu.VMEM((K,), jnp.float32),
            pltpu.VMEM((K,), jnp.float32),
        ],
        compiler_params=pltpu.CompilerParams(needs_layout_passes=False),
        interpret=interpret,
    )
    return f(attn, inputs)


def kernel(attn, inputs):
    return _run(attn, inputs)


# speedup vs baseline: 1.0621x; 1.0381x over previous
import functools

import jax
import jax.numpy as jnp
from jax import lax
from jax.experimental import pallas as pl
from jax.experimental.pallas import tpu as pltpu
from jax.experimental.pallas import tpu_sc as plsc

R = 128
N = 8192
K = 1024
L = 16
NC = 2
NS = 16
W = NC * NS
RPW = R // W
NV = N // L
BLK = N // L
UNROLL = 4

I32MAX = 0x7FFFFFFF


def _srl(x, n):
    return lax.shift_right_logical(x, n)


def _body(attn_hbm, inp_hbm, oattn_hbm, oinp_hbm,
          attn_v, inp_v, skey_v, ckey_a, cidx_a, ckey_b, cidx_b,
          hist256, hist32, hist32b, histbase, binsum, stag_a, stag_i):
    cid = lax.axis_index("c")
    sid = lax.axis_index("s")
    wid = sid * NC + cid
    lanes = lax.broadcasted_iota(jnp.int32, (L,), 0)
    zeros = jnp.zeros((L,), jnp.int32)
    ones = jnp.ones((L,), jnp.int32)
    lanebase = lanes * 256

    def z0(j, _):
        hist256[pl.ds(j * L, L)] = zeros
        return 0
    lax.fori_loop(0, 256, z0, 0)

    def z1(j, _):
        hist32[pl.ds(j * L, L)] = zeros
        hist32b[pl.ds(j * L, L)] = zeros
        return 0
    lax.fori_loop(0, 32, z1, 0)

    def byte_scan(threshold, save_binsum):
        def grp(j, carry):
            cum_carry, cntv = carry
            acc = zeros
            for l in range(L):
                sl = l * 256 + j * L
                acc = acc + hist256[pl.ds(sl, L)]
                hist256[pl.ds(sl, L)] = zeros
            inc = plsc.cumsum(acc) + cum_carry
            if save_binsum:
                binsum[pl.ds(j * L, L)] = inc
            cntv = cntv + jnp.where(inc < threshold, 1, 0)
            return (jnp.max(inc), cntv)
        _, cntv = lax.fori_loop(0, 16, grp, (jnp.int32(0), zeros))
        return jnp.sum(cntv)

    def do_row(i, _carry):
        r = wid * RPW + i
        pltpu.sync_copy(attn_hbm.at[r], attn_v)
        pltpu.sync_copy(inp_hbm.at[r], inp_v)

        def key_hist(t4, _):
            for u in range(UNROLL):
                t = t4 * UNROLL + u
                x = attn_v[pl.ds(t * L, L)]
                uu = lax.bitcast_convert_type(x, jnp.int32)
                sk = jnp.where(uu < 0, uu & I32MAX, ~uu)
                skey_v[pl.ds(t * L, L)] = sk
                d = (_srl(sk, 24) & 0xFF) ^ 0x80
                slot = lanebase + d
                plsc.addupdate_scatter(hist256, [slot], ones)
            return 0
        lax.fori_loop(0, NV // UNROLL, key_hist, 0)

        p1 = byte_scan(jnp.int32(K), True)
        bm1 = plsc.load_gather(binsum, [zeros + jnp.maximum(p1 - 1, 0)])
        below1 = jnp.where(p1 == 0, jnp.int32(0), jnp.max(bm1))

        def hist2(t4, _):
            for u in range(UNROLL):
                t = t4 * UNROLL + u
                sk = skey_v[pl.ds(t * L, L)]
                d1 = (_srl(sk, 24) & 0xFF) ^ 0x80
                m = d1 == p1
                slot = lanebase + (_srl(sk, 16) & 0xFF)
                plsc.addupdate_scatter(hist256, [slot], ones, mask=m)
            return 0
        lax.fori_loop(0, NV // UNROLL, hist2, 0)

        p2 = byte_scan(K - below1, False)

        skey_ub = (
            lax.shift_left((p1 ^ 0x80), 24)
            | lax.shift_left(p2, 16)
            | jnp.int32(0xFFFF))

        def compact(t4, off):
            for u in range(UNROLL):
                t = t4 * UNROLL + u
                pos = lanes * BLK + t
                s = plsc.load_gather(skey_v, [pos])
                m = s <= skey_ub
                plsc.store_scatter(ckey_a, [off], s, mask=m)
                plsc.store_scatter(cidx_a, [off], pos, mask=m)
                plsc.addupdate_scatter(
                    hist32, [(s & 0x1F) * L + lanes], ones, mask=m)
                off = off + jnp.where(m, 1, 0)
            return off
        off_fin = lax.fori_loop(0, BLK // UNROLL, compact, lanes * BLK)
        cnt = off_fin - lanes * BLK
        ncand = jnp.sum(cnt)
        cmax = jnp.max(cnt)
        c1 = (ncand + (L - 1)) // L

        hists = (hist32, hist32b)

        def scan32(tbl):
            def s(j, carry):
                v = tbl[pl.ds(j * L, L)]
                tbl[pl.ds(j * L, L)] = zeros
                inc = plsc.cumsum(v)
                histbase[pl.ds(j * L, L)] = inc - v + carry
                return carry + jnp.max(inc)
            lax.fori_loop(0, 32, s, jnp.int32(0))

        def sort_pass(p, src_k, src_i, dst_k, dst_i, span, stride, msk_cnt):
            tbl = hists[p % 2]
            sh = 5 * p
            flip = 2 if p == 6 else 0
            if p > 0:
                def hist_step(t, _):
                    pos = lanes * stride + t
                    k = plsc.load_gather(src_k, [pos])
                    d = (_srl(k, sh) & 0x1F) ^ flip
                    plsc.addupdate_scatter(tbl, [d * L + lanes], ones)
                    return 0
                lax.fori_loop(0, span, hist_step, 0)
            scan32(tbl)

            def perm_step(t, _):
                pos = lanes * stride + t
                k = plsc.load_gather(src_k, [pos])
                v = plsc.load_gather(src_i, [pos])
                d = (_srl(k, sh) & 0x1F) ^ flip
                slot = d * L + lanes
                o = plsc.load_gather(histbase, [slot])
                m = None if msk_cnt is None else (t < msk_cnt)
                plsc.store_scatter(histbase, [slot], o + 1, mask=m)
                if p == 6:
                    mo = o < K if m is None else (m & (o < K))
                    va = plsc.load_gather(attn_v, [v])
                    vi = plsc.load_gather(inp_v, [v])
                    plsc.store_scatter(stag_a, [o], va, mask=mo)
                    plsc.store_scatter(stag_i, [o], vi, mask=mo)
                else:
                    plsc.store_scatter(dst_k, [o], k, mask=m)
                    plsc.store_scatter(dst_i, [o], v, mask=m)
                return 0
            lax.fori_loop(0, span, perm_step, 0)

        sort_pass(0, ckey_a, cidx_a, ckey_b, cidx_b, cmax, BLK, cnt)

        padpos = ncand + lanes
        padm = padpos < c1 * L
        plsc.store_scatter(
            ckey_b, [padpos], jnp.full((L,), I32MAX, jnp.int32), mask=padm)
        plsc.store_scatter(cidx_b, [padpos], zeros, mask=padm)

        bufs = ((ckey_b, cidx_b), (ckey_a, cidx_a))
        for p in range(1, 7):
            src_k, src_i = bufs[(p - 1) % 2]
            dst_k, dst_i = bufs[p % 2]
            sort_pass(p, src_k, src_i, dst_k, dst_i, c1, c1, None)

        pltpu.sync_copy(stag_a, oattn_hbm.at[r])
        pltpu.sync_copy(stag_i, oinp_hbm.at[r])
        return 0

    lax.fori_loop(0, RPW, do_row, 0)


@functools.partial(jax.jit, static_argnames=("interpret",))
def _run(attn, inputs, interpret=False):
    mesh = plsc.VectorSubcoreMesh(
        core_axis_name="c", subcore_axis_name="s",
        num_cores=NC, num_subcores=NS)
    f = pl.kernel(
        _body,
        out_type=(
            jax.ShapeDtypeStruct((R, K), jnp.float32),
            jax.ShapeDtypeStruct((R, K), jnp.float32),
        ),
        mesh=mesh,
        scratch_types=[
            pltpu.VMEM((N,), jnp.float32),
            pltpu.VMEM((N,), jnp.float32),
            pltpu.VMEM((N,), jnp.int32),
            pltpu.VMEM((N,), jnp.int32),
            pltpu.VMEM((N,), jnp.int32),
            pltpu.VMEM((N,), jnp.int32),
            pltpu.VMEM((N,), jnp.int32),
            pltpu.VMEM((256 * L,), jnp.int32),
            pltpu.VMEM((32 * L,), jnp.int32),
            pltpu.VMEM((32 * L,), jnp.int32),
            pltpu.VMEM((32 * L,), jnp.int32),
            pltpu.VMEM((256,), jnp.int32),
            pltpu.VMEM((K,), jnp.float32),
            pltpu.VMEM((K,), jnp.float32),
        ],
        compiler_params=pltpu.CompilerParams(needs_layout_passes=False),
        interpret=interpret,
    )
    return f(attn, inputs)


def kernel(attn, inputs):
    return _run(attn, inputs)


# speedup vs baseline: 1.1236x; 1.0579x over previous
import functools

import jax
import jax.numpy as jnp
from jax import lax
from jax.experimental import pallas as pl
from jax.experimental.pallas import tpu as pltpu
from jax.experimental.pallas import tpu_sc as plsc

R = 128
N = 8192
K = 1024
L = 16
NC = 2
NS = 16
W = NC * NS
RPW = R // W
NV = N // L
BLK = N // L
UNROLL = 4

I32MAX = 0x7FFFFFFF


def _srl(x, n):
    return lax.shift_right_logical(x, n)


def _body(attn_hbm, inp_hbm, oattn_hbm, oinp_hbm,
          attn_v, inp_v, skey_v, ckey_a, cidx_a, ckey_b, cidx_b,
          hist256, hist32, hist32b, histbase, binsum, stag_a, stag_i):
    cid = lax.axis_index("c")
    sid = lax.axis_index("s")
    wid = sid * NC + cid
    lanes = lax.broadcasted_iota(jnp.int32, (L,), 0)
    zeros = jnp.zeros((L,), jnp.int32)
    ones = jnp.ones((L,), jnp.int32)
    lanebase = lanes * 256

    def z0(j, _):
        hist256[pl.ds(j * L, L)] = zeros
        return 0
    lax.fori_loop(0, 256, z0, 0)

    def z1(j, _):
        hist32[pl.ds(j * L, L)] = zeros
        hist32b[pl.ds(j * L, L)] = zeros
        return 0
    lax.fori_loop(0, 64, z1, 0)

    def byte_scan(threshold, save_binsum):
        def grp(j, carry):
            cum_carry, cntv = carry
            acc = zeros
            for l in range(L):
                sl = l * 256 + j * L
                acc = acc + hist256[pl.ds(sl, L)]
                hist256[pl.ds(sl, L)] = zeros
            inc = plsc.cumsum(acc) + cum_carry
            if save_binsum:
                binsum[pl.ds(j * L, L)] = inc
            cntv = cntv + jnp.where(inc < threshold, 1, 0)
            return (jnp.max(inc), cntv)
        _, cntv = lax.fori_loop(0, 16, grp, (jnp.int32(0), zeros))
        return jnp.sum(cntv)

    def do_row(i, _carry):
        r = wid * RPW + i
        pltpu.sync_copy(attn_hbm.at[r], attn_v)
        pltpu.sync_copy(inp_hbm.at[r], inp_v)

        def key_hist(t4, _):
            for u in range(UNROLL):
                t = t4 * UNROLL + u
                x = attn_v[pl.ds(t * L, L)]
                uu = lax.bitcast_convert_type(x, jnp.int32)
                sk = jnp.where(uu < 0, uu & I32MAX, ~uu)
                skey_v[pl.ds(t * L, L)] = sk
                d = (_srl(sk, 24) & 0xFF) ^ 0x80
                slot = lanebase + d
                plsc.addupdate_scatter(hist256, [slot], ones)
            return 0
        lax.fori_loop(0, NV // UNROLL, key_hist, 0)

        p1 = byte_scan(jnp.int32(K), True)
        bm1 = plsc.load_gather(binsum, [zeros + jnp.maximum(p1 - 1, 0)])
        below1 = jnp.where(p1 == 0, jnp.int32(0), jnp.max(bm1))

        def hist2(t4, _):
            for u in range(UNROLL):
                t = t4 * UNROLL + u
                sk = skey_v[pl.ds(t * L, L)]
                d1 = (_srl(sk, 24) & 0xFF) ^ 0x80
                m = d1 == p1
                slot = lanebase + (_srl(sk, 16) & 0xFF)
                plsc.addupdate_scatter(hist256, [slot], ones, mask=m)
            return 0
        lax.fori_loop(0, NV // UNROLL, hist2, 0)

        p2 = byte_scan(K - below1, False)

        skey_ub = (
            lax.shift_left((p1 ^ 0x80), 24)
            | lax.shift_left(p2, 16)
            | jnp.int32(0xFFFF))

        def compact(t4, off):
            for u in range(UNROLL):
                t = t4 * UNROLL + u
                pos = lanes * BLK + t
                s = plsc.load_gather(skey_v, [pos])
                m = s <= skey_ub
                plsc.store_scatter(ckey_a, [off], s, mask=m)
                plsc.store_scatter(cidx_a, [off], pos, mask=m)
                plsc.addupdate_scatter(
                    hist32, [(s & 0x3F) * L + lanes], ones, mask=m)
                off = off + jnp.where(m, 1, 0)
            return off
        off_fin = lax.fori_loop(0, BLK // UNROLL, compact, lanes * BLK)
        cnt = off_fin - lanes * BLK
        ncand = jnp.sum(cnt)
        cmax = jnp.max(cnt)
        c1 = (ncand + (L - 1)) // L

        SHIFTS = (0, 6, 11, 16, 21, 26)
        MASKS = (0x3F, 0x1F, 0x1F, 0x1F, 0x1F, 0x3F)
        NBINS = (64, 32, 32, 32, 32, 64)
        FLIPS = (0, 0, 0, 0, 0, 0x20)
        LAST = 5
        hists = (hist32, hist32b)

        def scan_tbl(tbl, nb):
            def s(j, carry):
                v = tbl[pl.ds(j * L, L)]
                tbl[pl.ds(j * L, L)] = zeros
                inc = plsc.cumsum(v)
                histbase[pl.ds(j * L, L)] = inc - v + carry
                return carry + jnp.max(inc)
            lax.fori_loop(0, nb, s, jnp.int32(0))

        def sort_pass(p, src_k, src_i, dst_k, dst_i, span, stride, msk_cnt):
            tbl = hists[p % 2]
            sh, msk, flip = SHIFTS[p], MASKS[p], FLIPS[p]
            if p > 0:
                def hist_step(t, _):
                    pos = lanes * stride + t
                    k = plsc.load_gather(src_k, [pos])
                    d = (_srl(k, sh) & msk) ^ flip
                    plsc.addupdate_scatter(tbl, [d * L + lanes], ones)
                    return 0
                lax.fori_loop(0, span, hist_step, 0)
            scan_tbl(tbl, NBINS[p])

            def perm_step(t, _):
                pos = lanes * stride + t
                k = plsc.load_gather(src_k, [pos])
                v = plsc.load_gather(src_i, [pos])
                d = (_srl(k, sh) & msk) ^ flip
                slot = d * L + lanes
                o = plsc.load_gather(histbase, [slot])
                m = None if msk_cnt is None else (t < msk_cnt)
                plsc.store_scatter(histbase, [slot], o + 1, mask=m)
                if p == LAST:
                    mo = o < K if m is None else (m & (o < K))
                    va = plsc.load_gather(attn_v, [v])
                    vi = plsc.load_gather(inp_v, [v])
                    plsc.store_scatter(stag_a, [o], va, mask=mo)
                    plsc.store_scatter(stag_i, [o], vi, mask=mo)
                else:
                    plsc.store_scatter(dst_k, [o], k, mask=m)
                    plsc.store_scatter(dst_i, [o], v, mask=m)
                return 0
            lax.fori_loop(0, span, perm_step, 0)

        sort_pass(0, ckey_a, cidx_a, ckey_b, cidx_b, cmax, BLK, cnt)

        padpos = ncand + lanes
        padm = padpos < c1 * L
        plsc.store_scatter(
            ckey_b, [padpos], jnp.full((L,), I32MAX, jnp.int32), mask=padm)
        plsc.store_scatter(cidx_b, [padpos], zeros, mask=padm)

        bufs = ((ckey_b, cidx_b), (ckey_a, cidx_a))
        for p in range(1, LAST + 1):
            src_k, src_i = bufs[(p - 1) % 2]
            dst_k, dst_i = bufs[p % 2]
            sort_pass(p, src_k, src_i, dst_k, dst_i, c1, c1, None)

        pltpu.sync_copy(stag_a, oattn_hbm.at[r])
        pltpu.sync_copy(stag_i, oinp_hbm.at[r])
        return 0

    lax.fori_loop(0, RPW, do_row, 0)


@functools.partial(jax.jit, static_argnames=("interpret",))
def _run(attn, inputs, interpret=False):
    mesh = plsc.VectorSubcoreMesh(
        core_axis_name="c", subcore_axis_name="s",
        num_cores=NC, num_subcores=NS)
    f = pl.kernel(
        _body,
        out_type=(
            jax.ShapeDtypeStruct((R, K), jnp.float32),
            jax.ShapeDtypeStruct((R, K), jnp.float32),
        ),
        mesh=mesh,
        scratch_types=[
            pltpu.VMEM((N,), jnp.float32),
            pltpu.VMEM((N,), jnp.float32),
            pltpu.VMEM((N,), jnp.int32),
            pltpu.VMEM((N,), jnp.int32),
            pltpu.VMEM((N,), jnp.int32),
            pltpu.VMEM((N,), jnp.int32),
            pltpu.VMEM((N,), jnp.int32),
            pltpu.VMEM((256 * L,), jnp.int32),
            pltpu.VMEM((64 * L,), jnp.int32),
            pltpu.VMEM((64 * L,), jnp.int32),
            pltpu.VMEM((64 * L,), jnp.int32),
            pltpu.VMEM((256,), jnp.int32),
            pltpu.VMEM((K,), jnp.float32),
            pltpu.VMEM((K,), jnp.float32),
        ],
        compiler_params=pltpu.CompilerParams(needs_layout_passes=False),
        interpret=interpret,
    )
    return f(attn, inputs)


def kernel(attn, inputs):
    return _run(attn, inputs)


# speedup vs baseline: 1.2586x; 1.1202x over previous
import functools

import jax
import jax.numpy as jnp
from jax import lax
from jax.experimental import pallas as pl
from jax.experimental.pallas import tpu as pltpu
from jax.experimental.pallas import tpu_sc as plsc

R = 128
N = 8192
K = 1024
L = 16
NC = 2
NS = 16
W = NC * NS
RPW = R // W
NV = N // L
BLK = N // L
UNROLL = 4

I32MAX = 0x7FFFFFFF


def _srl(x, n):
    return lax.shift_right_logical(x, n)


def _body(attn_hbm, inp_hbm, oattn_hbm, oinp_hbm,
          attn_v, inp_v, skey_v, ckey_a, cidx_a, ckey_b, cidx_b,
          hist256, hist256b, hist32, hist32b, histbase, binsum, stag_a, stag_i):
    cid = lax.axis_index("c")
    sid = lax.axis_index("s")
    wid = sid * NC + cid
    lanes = lax.broadcasted_iota(jnp.int32, (L,), 0)
    zeros = jnp.zeros((L,), jnp.int32)
    ones = jnp.ones((L,), jnp.int32)
    lanebase = lanes * 256

    def z0(j, _):
        hist256[pl.ds(j * L, L)] = zeros
        hist256b[pl.ds(j * L, L)] = zeros
        return 0
    lax.fori_loop(0, 256, z0, 0)

    def z1(j, _):
        hist32[pl.ds(j * L, L)] = zeros
        hist32b[pl.ds(j * L, L)] = zeros
        return 0
    lax.fori_loop(0, 64, z1, 0)

    def byte_scan(tbl, threshold, save_binsum):
        def grp(j, carry):
            cum_carry, cntv = carry
            acc = zeros
            for l in range(L):
                sl = l * 256 + j * L
                acc = acc + tbl[pl.ds(sl, L)]
                tbl[pl.ds(sl, L)] = zeros
            inc = plsc.cumsum(acc) + cum_carry
            if save_binsum:
                binsum[pl.ds(j * L, L)] = inc
            cntv = cntv + jnp.where(inc < threshold, 1, 0)
            return (jnp.max(inc), cntv)
        _, cntv = lax.fori_loop(0, 16, grp, (jnp.int32(0), zeros))
        return jnp.sum(cntv)

    def do_row(i, _carry):
        r = wid * RPW + i
        pltpu.sync_copy(attn_hbm.at[r], attn_v)
        pltpu.sync_copy(inp_hbm.at[r], inp_v)

        PRED = 64
        def key_hist(t4, _):
            for u in range(UNROLL):
                t = t4 * UNROLL + u
                x = attn_v[pl.ds(t * L, L)]
                uu = lax.bitcast_convert_type(x, jnp.int32)
                sk = jnp.where(uu < 0, uu & I32MAX, ~uu)
                skey_v[pl.ds(t * L, L)] = sk
                d = (_srl(sk, 24) & 0xFF) ^ 0x80
                slot = lanebase + d
                plsc.addupdate_scatter(hist256, [slot], ones)
                slot2 = lanebase + (_srl(sk, 16) & 0xFF)
                plsc.addupdate_scatter(hist256b, [slot2], ones, mask=d == PRED)
            return 0
        lax.fori_loop(0, NV // UNROLL, key_hist, 0)

        p1 = byte_scan(hist256, jnp.int32(K), True)
        bm1 = plsc.load_gather(binsum, [zeros + jnp.maximum(p1 - 1, 0)])
        below1 = jnp.where(p1 == 0, jnp.int32(0), jnp.max(bm1))

        @pl.when(p1 != PRED)
        def _fallback():
            def zb(j, _):
                hist256b[pl.ds(j * L, L)] = zeros
                return 0
            lax.fori_loop(0, 256, zb, 0)

            def hist2(t4, _):
                for u in range(UNROLL):
                    t = t4 * UNROLL + u
                    sk = skey_v[pl.ds(t * L, L)]
                    d1 = (_srl(sk, 24) & 0xFF) ^ 0x80
                    m = d1 == p1
                    slot = lanebase + (_srl(sk, 16) & 0xFF)
                    plsc.addupdate_scatter(hist256b, [slot], ones, mask=m)
                return 0
            lax.fori_loop(0, NV // UNROLL, hist2, 0)

        p2 = byte_scan(hist256b, K - below1, False)

        skey_ub = (
            lax.shift_left((p1 ^ 0x80), 24)
            | lax.shift_left(p2, 16)
            | jnp.int32(0xFFFF))

        def compact(t4, off):
            for u in range(UNROLL):
                t = t4 * UNROLL + u
                pos = lanes * BLK + t
                s = plsc.load_gather(skey_v, [pos])
                m = s <= skey_ub
                plsc.store_scatter(ckey_a, [off], s, mask=m)
                plsc.store_scatter(cidx_a, [off], pos, mask=m)
                plsc.addupdate_scatter(
                    hist32, [(s & 0x3F) * L + lanes], ones, mask=m)
                off = off + jnp.where(m, 1, 0)
            return off
        off_fin = lax.fori_loop(0, BLK // UNROLL, compact, lanes * BLK)
        cnt = off_fin - lanes * BLK
        ncand = jnp.sum(cnt)
        cmax = jnp.max(cnt)
        c1 = (ncand + (L - 1)) // L

        SHIFTS = (0, 6, 11, 16, 21, 26)
        MASKS = (0x3F, 0x1F, 0x1F, 0x1F, 0x1F, 0x3F)
        NBINS = (64, 32, 32, 32, 32, 64)
        FLIPS = (0, 0, 0, 0, 0, 0x20)
        LAST = 5
        hists = (hist32, hist32b)

        def scan_tbl(tbl, nb):
            def s(j, carry):
                v = tbl[pl.ds(j * L, L)]
                tbl[pl.ds(j * L, L)] = zeros
                inc = plsc.cumsum(v)
                histbase[pl.ds(j * L, L)] = inc - v + carry
                return carry + jnp.max(inc)
            lax.fori_loop(0, nb, s, jnp.int32(0))

        def sort_pass(p, src_k, src_i, dst_k, dst_i, span, stride, msk_cnt):
            tbl = hists[p % 2]
            sh, msk, flip = SHIFTS[p], MASKS[p], FLIPS[p]
            if p > 0:
                def hist_step(t, _):
                    pos = lanes * stride + t
                    k = plsc.load_gather(src_k, [pos])
                    d = (_srl(k, sh) & msk) ^ flip
                    plsc.addupdate_scatter(tbl, [d * L + lanes], ones)
                    return 0
                lax.fori_loop(0, span, hist_step, 0)
            scan_tbl(tbl, NBINS[p])

            def perm_step(t, _):
                pos = lanes * stride + t
                k = plsc.load_gather(src_k, [pos])
                v = plsc.load_gather(src_i, [pos])
                d = (_srl(k, sh) & msk) ^ flip
                slot = d * L + lanes
                o = plsc.load_gather(histbase, [slot])
                m = None if msk_cnt is None else (t < msk_cnt)
                plsc.store_scatter(histbase, [slot], o + 1, mask=m)
                if p == LAST:
                    mo = o < K if m is None else (m & (o < K))
                    va = plsc.load_gather(attn_v, [v])
                    vi = plsc.load_gather(inp_v, [v])
                    plsc.store_scatter(stag_a, [o], va, mask=mo)
                    plsc.store_scatter(stag_i, [o], vi, mask=mo)
                else:
                    plsc.store_scatter(dst_k, [o], k, mask=m)
                    plsc.store_scatter(dst_i, [o], v, mask=m)
                return 0
            lax.fori_loop(0, span, perm_step, 0)

        sort_pass(0, ckey_a, cidx_a, ckey_b, cidx_b, cmax, BLK, cnt)

        padpos = ncand + lanes
        padm = padpos < c1 * L
        plsc.store_scatter(
            ckey_b, [padpos], jnp.full((L,), I32MAX, jnp.int32), mask=padm)
        plsc.store_scatter(cidx_b, [padpos], zeros, mask=padm)

        bufs = ((ckey_b, cidx_b), (ckey_a, cidx_a))
        for p in range(1, LAST + 1):
            src_k, src_i = bufs[(p - 1) % 2]
            dst_k, dst_i = bufs[p % 2]
            sort_pass(p, src_k, src_i, dst_k, dst_i, c1, c1, None)

        pltpu.sync_copy(stag_a, oattn_hbm.at[r])
        pltpu.sync_copy(stag_i, oinp_hbm.at[r])
        return 0

    lax.fori_loop(0, RPW, do_row, 0)


@functools.partial(jax.jit, static_argnames=("interpret",))
def _run(attn, inputs, interpret=False):
    mesh = plsc.VectorSubcoreMesh(
        core_axis_name="c", subcore_axis_name="s",
        num_cores=NC, num_subcores=NS)
    f = pl.kernel(
        _body,
        out_type=(
            jax.ShapeDtypeStruct((R, K), jnp.float32),
            jax.ShapeDtypeStruct((R, K), jnp.float32),
        ),
        mesh=mesh,
        scratch_types=[
            pltpu.VMEM((N,), jnp.float32),
            pltpu.VMEM((N,), jnp.float32),
            pltpu.VMEM((N,), jnp.int32),
            pltpu.VMEM((N,), jnp.int32),
            pltpu.VMEM((N,), jnp.int32),
            pltpu.VMEM((N,), jnp.int32),
            pltpu.VMEM((N,), jnp.int32),
            pltpu.VMEM((256 * L,), jnp.int32),
            pltpu.VMEM((256 * L,), jnp.int32),
            pltpu.VMEM((64 * L,), jnp.int32),
            pltpu.VMEM((64 * L,), jnp.int32),
            pltpu.VMEM((64 * L,), jnp.int32),
            pltpu.VMEM((256,), jnp.int32),
            pltpu.VMEM((K,), jnp.float32),
            pltpu.VMEM((K,), jnp.float32),
        ],
        compiler_params=pltpu.CompilerParams(needs_layout_passes=False),
        interpret=interpret,
    )
    return f(attn, inputs)


def kernel(attn, inputs):
    return _run(attn, inputs)
